# baseline (speedup 1.0000x reference)
import functools

import jax
import jax.numpy as jnp
from jax import lax
from jax.experimental import pallas as pl
from jax.experimental.pallas import tpu as pltpu
from jax.experimental.pallas import tpu_sc as plsc

_D = 32
_B = 16384
_MARGIN = 1.0
_L = 16
_NW = 32
_BW = _B // _NW
_CH = 128
_NCH = _BW // _CH
_NVC = _CH // _L

_mesh = plsc.VectorSubcoreMesh(
    core_axis_name="c", subcore_axis_name="s", num_cores=2, num_subcores=16
)


@functools.partial(
    pl.kernel,
    out_type=jax.ShapeDtypeStruct((_NW, _L), jnp.float32),
    mesh=_mesh,
    scratch_types=(
        [pltpu.VMEM((_NCH, _CH), jnp.int32) for _ in range(6)]
        + [pltpu.VMEM((_BW,), jnp.int32) for _ in range(6)]
        + [pltpu.VMEM((_CH, _CH), jnp.float32) for _ in range(6)]
        + [pltpu.VMEM((_L,), jnp.float32), pltpu.SemaphoreType.DMA]
    ),
    compiler_params=pltpu.CompilerParams(
        needs_layout_passes=False, use_tc_tiling_on_sc=True
    ),
)
def _transe_sc(ent_hbm, rel_hbm,
               p0, p1, p2, p3, p4, p5,
               s0, s1, s2, s3, s4, s5,
               out_hbm,
               ip0, ip1, ip2, ip3, ip4, ip5,
               is0, is1, is2, is3, is4, is5,
               b0, b1, b2, b3, b4, b5,
               loss_v, sem):
    wid = lax.axis_index("s") * 2 + lax.axis_index("c")

    p_hbms = (p0, p1, p2, p3, p4, p5)
    s_hbms = (s0, s1, s2, s3, s4, s5)
    ip_refs = (ip0, ip1, ip2, ip3, ip4, ip5)
    is_refs = (is0, is1, is2, is3, is4, is5)
    bufs = (b0, b1, b2, b3, b4, b5)
    tables = (ent_hbm, rel_hbm, ent_hbm, ent_hbm, rel_hbm, ent_hbm)

    copies = [pltpu.async_copy(h.at[wid], r, sem)
              for h, r in zip(p_hbms + s_hbms, ip_refs + is_refs)]
    for c in copies:
        c.wait()

    lane = lax.iota(jnp.int32, _L)
    zero = lax.broadcast(jnp.float32(0.0), (_L,))
    loss = zero

    for c in range(_NCH):
        gathers = [
            pltpu.async_copy(tab.at[iref.at[c]], bref, sem)
            for tab, iref, bref in zip(tables, ip_refs, bufs)
        ]
        for g in gathers:
            g.wait()

        def vreg_body(v, loss_sum, _c=c):
            row = lane + v * _L
            off = _c * _CH
            cols = [plsc.load_gather(sref, [row + off]) for sref in is_refs]
            acc_p = zero
            acc_n = zero
            for d in range(_D):
                hp = plsc.load_gather(b0, [row, cols[0] + d])
                rp = plsc.load_gather(b1, [row, cols[1] + d])
                tp = plsc.load_gather(b2, [row, cols[2] + d])
                acc_p = acc_p + jnp.abs(hp + rp - tp)
                hn = plsc.load_gather(b3, [row, cols[3] + d])
                rn = plsc.load_gather(b4, [row, cols[4] + d])
                tn = plsc.load_gather(b5, [row, cols[5] + d])
                acc_n = acc_n + jnp.abs(hn + rn - tn)
            hinge = jnp.maximum(acc_p - acc_n + jnp.float32(_MARGIN), zero)
            return loss_sum + hinge

        loss = lax.fori_loop(0, _NVC, vreg_body, loss)

    loss_v[...] = loss
    pltpu.sync_copy(loss_v, out_hbm.at[wid])


_NE = 1000000
_RB = 512
_NRB = -(-_NE // _RB)
_NPR = _NRB * (_RB // 4)


def _relayout_body(in_ref, out_ref):
    y = in_ref[...].T
    for q in range(4):
        out_ref[:, q * _D:(q + 1) * _D] = y[q * 128:(q + 1) * 128, :]


def _pack_rows(table_t):
    return pl.pallas_call(
        _relayout_body,
        grid=(_NRB,),
        in_specs=[pl.BlockSpec((_D, _RB), lambda i: (0, i))],
        out_specs=pl.BlockSpec((_RB // 4, 4 * _D), lambda i: (i, 0)),
        out_shape=jax.ShapeDtypeStruct((_NPR, 4 * _D), jnp.float32),
    )(table_t)


def kernel(positive_triplets, negative_triplets, entity_emb, relation_emb):
    packed = []
    offs = []
    for arr in (positive_triplets, negative_triplets):
        for c in range(3):
            col = arr[:, c]
            packed.append(((col >> 9) * 128 + (col & 127)).reshape(_NW, _NCH, _CH))
            offs.append((((col >> 7) & 3) * _D).reshape(_NW, _BW))
    partials = _transe_sc(
        _pack_rows(entity_emb.T),
        _pack_rows(relation_emb.T),
        *packed, *offs,
    )
    return jnp.sum(partials) / jnp.float32(_B)


# speedup vs baseline: 3.5190x; 3.5190x over previous
import functools

import jax
import jax.numpy as jnp
from jax import lax
from jax.experimental import pallas as pl
from jax.experimental.pallas import tpu as pltpu
from jax.experimental.pallas import tpu_sc as plsc

_D = 32
_B = 16384
_MARGIN = 1.0
_L = 16
_NW = 32
_BW = _B // _NW
_CH = 128
_NCH = _BW // _CH
_NVC = _CH // _L

_mesh = plsc.VectorSubcoreMesh(
    core_axis_name="c", subcore_axis_name="s", num_cores=2, num_subcores=16
)


@functools.partial(
    pl.kernel,
    out_type=jax.ShapeDtypeStruct((_NW, _L), jnp.float32),
    mesh=_mesh,
    scratch_types=(
        [pltpu.VMEM((_NCH, _CH), jnp.int32) for _ in range(6)]
        + [pltpu.VMEM((_BW,), jnp.int32) for _ in range(6)]
        + [pltpu.VMEM((_CH, _CH), jnp.float32) for _ in range(6)]
        + [pltpu.VMEM((_L,), jnp.float32), pltpu.SemaphoreType.DMA]
    ),
    compiler_params=pltpu.CompilerParams(
        needs_layout_passes=False, use_tc_tiling_on_sc=True
    ),
)
def _transe_sc(ent_hbm, rel_hbm,
               p0, p1, p2, p3, p4, p5,
               s0, s1, s2, s3, s4, s5,
               out_hbm,
               ip0, ip1, ip2, ip3, ip4, ip5,
               is0, is1, is2, is3, is4, is5,
               b0, b1, b2, b3, b4, b5,
               loss_v, sem):
    wid = lax.axis_index("s") * 2 + lax.axis_index("c")

    p_hbms = (p0, p1, p2, p3, p4, p5)
    s_hbms = (s0, s1, s2, s3, s4, s5)
    ip_refs = (ip0, ip1, ip2, ip3, ip4, ip5)
    is_refs = (is0, is1, is2, is3, is4, is5)
    bufs = (b0, b1, b2, b3, b4, b5)
    tables = (ent_hbm, rel_hbm, ent_hbm, ent_hbm, rel_hbm, ent_hbm)

    copies = [pltpu.async_copy(h.at[wid], r, sem)
              for h, r in zip(p_hbms + s_hbms, ip_refs + is_refs)]
    for c in copies:
        c.wait()

    lane = lax.iota(jnp.int32, _L)
    zero = lax.broadcast(jnp.float32(0.0), (_L,))
    loss = zero

    for c in range(_NCH):
        gathers = [
            pltpu.async_copy(tab.at[iref.at[c]], bref, sem)
            for tab, iref, bref in zip(tables, ip_refs, bufs)
        ]
        for g in gathers:
            g.wait()

        def vreg_body(v, loss_sum, _c=c):
            row = lane + v * _L
            off = _c * _CH
            cols = [plsc.load_gather(sref, [row + off]) for sref in is_refs]
            acc_p = zero
            acc_n = zero
            for d in range(_D):
                hp = plsc.load_gather(b0, [row, cols[0] + d])
                rp = plsc.load_gather(b1, [row, cols[1] + d])
                tp = plsc.load_gather(b2, [row, cols[2] + d])
                acc_p = acc_p + jnp.abs(hp + rp - tp)
                hn = plsc.load_gather(b3, [row, cols[3] + d])
                rn = plsc.load_gather(b4, [row, cols[4] + d])
                tn = plsc.load_gather(b5, [row, cols[5] + d])
                acc_n = acc_n + jnp.abs(hn + rn - tn)
            hinge = jnp.maximum(acc_p - acc_n + jnp.float32(_MARGIN), zero)
            return loss_sum + hinge

        loss = lax.fori_loop(0, _NVC, vreg_body, loss)

    loss_v[...] = loss
    pltpu.sync_copy(loss_v, out_hbm.at[wid])


_NE = 1000000
_RB = 4096
_NRB = -(-_NE // _RB)
_NPR = _NRB * (_RB // 4)


def _relayout_body(in_ref, out_ref):
    x = in_ref[...]
    eye = jnp.eye(_D, dtype=jnp.float32)
    y = lax.dot_general(x, eye, (((0,), (0,)), ((), ())),
                        preferred_element_type=jnp.float32)
    for k in range(_RB // 512):
        for q in range(4):
            out_ref[k * 128:(k + 1) * 128, q * _D:(q + 1) * _D] = (
                y[k * 512 + q * 128:k * 512 + (q + 1) * 128, :]
            )


def _pack_rows(table_t):
    return pl.pallas_call(
        _relayout_body,
        grid=(_NRB,),
        in_specs=[pl.BlockSpec((_D, _RB), lambda i: (0, i))],
        out_specs=pl.BlockSpec((_RB // 4, 4 * _D), lambda i: (i, 0)),
        out_shape=jax.ShapeDtypeStruct((_NPR, 4 * _D), jnp.float32),
    )(table_t)


def kernel(positive_triplets, negative_triplets, entity_emb, relation_emb):
    packed = []
    offs = []
    for arr in (positive_triplets, negative_triplets):
        for c in range(3):
            col = arr[:, c]
            packed.append(((col >> 9) * 128 + (col & 127)).reshape(_NW, _NCH, _CH))
            offs.append((((col >> 7) & 3) * _D).reshape(_NW, _BW))
    partials = _transe_sc(
        _pack_rows(entity_emb.T),
        _pack_rows(relation_emb.T),
        *packed, *offs,
    )
    return jnp.sum(partials) / jnp.float32(_B)


# speedup vs baseline: 4.0818x; 1.1599x over previous
import functools

import jax
import jax.numpy as jnp
from jax import lax
from jax.experimental import pallas as pl
from jax.experimental.pallas import tpu as pltpu
from jax.experimental.pallas import tpu_sc as plsc

_D = 32
_B = 16384
_MARGIN = 1.0
_L = 16
_NW = 32
_BW = _B // _NW
_CH = 128
_NCH = _BW // _CH
_NVC = _CH // _L

_mesh = plsc.VectorSubcoreMesh(
    core_axis_name="c", subcore_axis_name="s", num_cores=2, num_subcores=16
)


@functools.partial(
    pl.kernel,
    out_type=jax.ShapeDtypeStruct((_NW, _L), jnp.float32),
    mesh=_mesh,
    scratch_types=(
        [pltpu.VMEM((_NCH, _CH), jnp.int32) for _ in range(6)]
        + [pltpu.VMEM((_BW,), jnp.int32) for _ in range(6)]
        + [pltpu.VMEM((_CH, _CH), jnp.float32) for _ in range(6)]
        + [pltpu.VMEM((_L,), jnp.float32), pltpu.SemaphoreType.DMA]
    ),
    compiler_params=pltpu.CompilerParams(
        needs_layout_passes=False, use_tc_tiling_on_sc=True
    ),
)
def _transe_sc(ent_hbm, rel_hbm,
               p0, p1, p2, p3, p4, p5,
               s0, s1, s2, s3, s4, s5,
               out_hbm,
               ip0, ip1, ip2, ip3, ip4, ip5,
               is0, is1, is2, is3, is4, is5,
               b0, b1, b2, b3, b4, b5,
               loss_v, sem):
    wid = lax.axis_index("s") * 2 + lax.axis_index("c")

    p_hbms = (p0, p1, p2, p3, p4, p5)
    s_hbms = (s0, s1, s2, s3, s4, s5)
    ip_refs = (ip0, ip1, ip2, ip3, ip4, ip5)
    is_refs = (is0, is1, is2, is3, is4, is5)
    bufs = (b0, b1, b2, b3, b4, b5)
    tables = (ent_hbm, rel_hbm, ent_hbm, ent_hbm, rel_hbm, ent_hbm)

    copies = [pltpu.async_copy(h.at[wid], r, sem)
              for h, r in zip(p_hbms + s_hbms, ip_refs + is_refs)]
    for c in copies:
        c.wait()

    lane = lax.iota(jnp.int32, _L)
    zero = lax.broadcast(jnp.float32(0.0), (_L,))
    loss = zero

    for c in range(_NCH):
        gathers = [
            pltpu.async_copy(tab.at[iref.at[c]], bref, sem)
            for tab, iref, bref in zip(tables, ip_refs, bufs)
        ]
        for g in gathers:
            g.wait()

        def vreg_body(v, loss_sum, _c=c):
            row = lane + v * _L
            off = _c * _CH
            cols = [plsc.load_gather(sref, [row + off]) for sref in is_refs]
            acc_p = zero
            acc_n = zero
            for d in range(_D):
                hp = plsc.load_gather(b0, [row, cols[0] + d])
                rp = plsc.load_gather(b1, [row, cols[1] + d])
                tp = plsc.load_gather(b2, [row, cols[2] + d])
                acc_p = acc_p + jnp.abs(hp + rp - tp)
                hn = plsc.load_gather(b3, [row, cols[3] + d])
                rn = plsc.load_gather(b4, [row, cols[4] + d])
                tn = plsc.load_gather(b5, [row, cols[5] + d])
                acc_n = acc_n + jnp.abs(hn + rn - tn)
            hinge = jnp.maximum(acc_p - acc_n + jnp.float32(_MARGIN), zero)
            return loss_sum + hinge

        loss = lax.fori_loop(0, _NVC, vreg_body, loss)

    loss_v[...] = loss
    pltpu.sync_copy(loss_v, out_hbm.at[wid])


_NE = 1000000
_RB = 16384
_NRB = -(-_NE // _RB)
_NPR = _NRB * (_RB // 4)


def _relayout_body(in_ref, out_ref):
    x = in_ref[...]
    eye = jnp.eye(_D, dtype=jnp.float32)
    y = lax.dot_general(x, eye, (((0,), (0,)), ((), ())),
                        preferred_element_type=jnp.float32)
    for k in range(_RB // 512):
        for q in range(4):
            out_ref[k * 128:(k + 1) * 128, q * _D:(q + 1) * _D] = (
                y[k * 512 + q * 128:k * 512 + (q + 1) * 128, :]
            )


def _pack_rows(table_t):
    return pl.pallas_call(
        _relayout_body,
        grid=(_NRB,),
        in_specs=[pl.BlockSpec((_D, _RB), lambda i: (0, i))],
        out_specs=pl.BlockSpec((_RB // 4, 4 * _D), lambda i: (i, 0)),
        out_shape=jax.ShapeDtypeStruct((_NPR, 4 * _D), jnp.float32),
    )(table_t)


def kernel(positive_triplets, negative_triplets, entity_emb, relation_emb):
    packed = []
    offs = []
    for arr in (positive_triplets, negative_triplets):
        for c in range(3):
            col = arr[:, c]
            packed.append(((col >> 9) * 128 + (col & 127)).reshape(_NW, _NCH, _CH))
            offs.append((((col >> 7) & 3) * _D).reshape(_NW, _BW))
    partials = _transe_sc(
        _pack_rows(entity_emb.T),
        _pack_rows(relation_emb.T),
        *packed, *offs,
    )
    return jnp.sum(partials) / jnp.float32(_B)


# speedup vs baseline: 5.2606x; 1.2888x over previous
import functools

import jax
import jax.numpy as jnp
from jax import lax
from jax.experimental import pallas as pl
from jax.experimental.pallas import tpu as pltpu
from jax.experimental.pallas import tpu_sc as plsc

_D = 32
_B = 16384
_MARGIN = 1.0
_L = 16
_NW = 32
_BW = _B // _NW
_CH = 128
_NCH = _BW // _CH
_NVC = _CH // _L

_mesh = plsc.VectorSubcoreMesh(
    core_axis_name="c", subcore_axis_name="s", num_cores=2, num_subcores=16
)


@functools.partial(
    pl.kernel,
    out_type=jax.ShapeDtypeStruct((_NW, _L), jnp.float32),
    mesh=_mesh,
    scratch_types=(
        [pltpu.VMEM((_NCH, _CH), jnp.int32) for _ in range(6)]
        + [pltpu.VMEM((_BW,), jnp.int32) for _ in range(6)]
        + [pltpu.VMEM((_CH, _CH), jnp.float32) for _ in range(6)]
        + [pltpu.VMEM((_L,), jnp.float32), pltpu.SemaphoreType.DMA]
    ),
    compiler_params=pltpu.CompilerParams(
        needs_layout_passes=False, use_tc_tiling_on_sc=True
    ),
)
def _transe_sc(ent_hbm, rel_hbm,
               p0, p1, p2, p3, p4, p5,
               s0, s1, s2, s3, s4, s5,
               out_hbm,
               ip0, ip1, ip2, ip3, ip4, ip5,
               is0, is1, is2, is3, is4, is5,
               b0, b1, b2, b3, b4, b5,
               loss_v, sem):
    wid = lax.axis_index("s") * 2 + lax.axis_index("c")

    p_hbms = (p0, p1, p2, p3, p4, p5)
    s_hbms = (s0, s1, s2, s3, s4, s5)
    ip_refs = (ip0, ip1, ip2, ip3, ip4, ip5)
    is_refs = (is0, is1, is2, is3, is4, is5)
    bufs = (b0, b1, b2, b3, b4, b5)
    tables = (ent_hbm, rel_hbm, ent_hbm, ent_hbm, rel_hbm, ent_hbm)

    copies = [pltpu.async_copy(h.at[wid], r, sem)
              for h, r in zip(p_hbms + s_hbms, ip_refs + is_refs)]
    for c in copies:
        c.wait()

    lane = lax.iota(jnp.int32, _L)
    zero = lax.broadcast(jnp.float32(0.0), (_L,))
    loss = zero

    for c in range(_NCH):
        gathers = [
            pltpu.async_copy(tab.at[iref.at[c]], bref, sem)
            for tab, iref, bref in zip(tables, ip_refs, bufs)
        ]
        for g in gathers:
            g.wait()

        def vreg_body(v, loss_sum, _c=c):
            row = lane + v * _L
            off = _c * _CH
            cols = [plsc.load_gather(sref, [row + off]) for sref in is_refs]
            acc_p = zero
            acc_n = zero
            for d in range(_D):
                hp = plsc.load_gather(b0, [row, cols[0] + d])
                rp = plsc.load_gather(b1, [row, cols[1] + d])
                tp = plsc.load_gather(b2, [row, cols[2] + d])
                acc_p = acc_p + jnp.abs(hp + rp - tp)
                hn = plsc.load_gather(b3, [row, cols[3] + d])
                rn = plsc.load_gather(b4, [row, cols[4] + d])
                tn = plsc.load_gather(b5, [row, cols[5] + d])
                acc_n = acc_n + jnp.abs(hn + rn - tn)
            hinge = jnp.maximum(acc_p - acc_n + jnp.float32(_MARGIN), zero)
            return loss_sum + hinge

        loss = lax.fori_loop(0, _NVC, vreg_body, loss)

    loss_v[...] = loss
    pltpu.sync_copy(loss_v, out_hbm.at[wid])


_NE = 1000000
_RB = 16384
_NRB = -(-_NE // _RB)
_NPR = _NRB * (_RB // 4)


def _relayout_body(in_ref, out_ref):
    x = in_ref[...].astype(jnp.bfloat16)
    eye = jnp.eye(_D, dtype=jnp.bfloat16)
    y = lax.dot_general(x, eye, (((0,), (0,)), ((), ())),
                        preferred_element_type=jnp.float32)
    for k in range(_RB // 512):
        for q in range(4):
            out_ref[k * 128:(k + 1) * 128, q * _D:(q + 1) * _D] = (
                y[k * 512 + q * 128:k * 512 + (q + 1) * 128, :]
            )


def _pack_rows(table_t):
    return pl.pallas_call(
        _relayout_body,
        grid=(_NRB,),
        in_specs=[pl.BlockSpec((_D, _RB), lambda i: (0, i))],
        out_specs=pl.BlockSpec((_RB // 4, 4 * _D), lambda i: (i, 0)),
        out_shape=jax.ShapeDtypeStruct((_NPR, 4 * _D), jnp.float32),
    )(table_t)


def kernel(positive_triplets, negative_triplets, entity_emb, relation_emb):
    packed = []
    offs = []
    for arr in (positive_triplets, negative_triplets):
        for c in range(3):
            col = arr[:, c]
            packed.append(((col >> 9) * 128 + (col & 127)).reshape(_NW, _NCH, _CH))
            offs.append((((col >> 7) & 3) * _D).reshape(_NW, _BW))
    partials = _transe_sc(
        _pack_rows(entity_emb.T),
        _pack_rows(relation_emb.T),
        *packed, *offs,
    )
    return jnp.sum(partials) / jnp.float32(_B)


# speedup vs baseline: 5.3083x; 1.0091x over previous
import functools

import jax
import jax.numpy as jnp
from jax import lax
from jax.experimental import pallas as pl
from jax.experimental.pallas import tpu as pltpu
from jax.experimental.pallas import tpu_sc as plsc

_D = 32
_B = 16384
_MARGIN = 1.0
_L = 16
_NW = 32
_BW = _B // _NW
_CH = 128
_NCH = _BW // _CH
_NVC = _CH // _L

_mesh = plsc.VectorSubcoreMesh(
    core_axis_name="c", subcore_axis_name="s", num_cores=2, num_subcores=16
)


@functools.partial(
    pl.kernel,
    out_type=jax.ShapeDtypeStruct((_NW, _L), jnp.float32),
    mesh=_mesh,
    scratch_types=(
        [pltpu.VMEM((_NCH, _CH), jnp.int32) for _ in range(6)]
        + [pltpu.VMEM((_BW,), jnp.int32) for _ in range(6)]
        + [pltpu.VMEM((_CH, _CH), jnp.float32) for _ in range(6)]
        + [pltpu.VMEM((_L,), jnp.float32), pltpu.SemaphoreType.DMA]
    ),
    compiler_params=pltpu.CompilerParams(
        needs_layout_passes=False, use_tc_tiling_on_sc=True
    ),
)
def _transe_sc(ent_hbm, rel_hbm,
               p0, p1, p2, p3, p4, p5,
               s0, s1, s2, s3, s4, s5,
               out_hbm,
               ip0, ip1, ip2, ip3, ip4, ip5,
               is0, is1, is2, is3, is4, is5,
               b0, b1, b2, b3, b4, b5,
               loss_v, sem):
    wid = lax.axis_index("s") * 2 + lax.axis_index("c")

    p_hbms = (p0, p1, p2, p3, p4, p5)
    s_hbms = (s0, s1, s2, s3, s4, s5)
    ip_refs = (ip0, ip1, ip2, ip3, ip4, ip5)
    is_refs = (is0, is1, is2, is3, is4, is5)
    bufs = (b0, b1, b2, b3, b4, b5)
    tables = (ent_hbm, rel_hbm, ent_hbm, ent_hbm, rel_hbm, ent_hbm)

    copies = [pltpu.async_copy(h.at[wid], r, sem)
              for h, r in zip(p_hbms + s_hbms, ip_refs + is_refs)]
    for c in copies:
        c.wait()

    lane = lax.iota(jnp.int32, _L)
    zero = lax.broadcast(jnp.float32(0.0), (_L,))
    loss = zero

    for c in range(_NCH):
        gathers = [
            pltpu.async_copy(tab.at[iref.at[c]], bref, sem)
            for tab, iref, bref in zip(tables, ip_refs, bufs)
        ]
        for g in gathers:
            g.wait()

        def vreg_body(v, loss_sum, _c=c):
            row = lane + v * _L
            off = _c * _CH
            cols = [plsc.load_gather(sref, [row + off]) for sref in is_refs]
            acc_p = zero
            acc_n = zero
            for d in range(_D):
                hp = plsc.load_gather(b0, [row, cols[0] + d])
                rp = plsc.load_gather(b1, [row, cols[1] + d])
                tp = plsc.load_gather(b2, [row, cols[2] + d])
                acc_p = acc_p + jnp.abs(hp + rp - tp)
                hn = plsc.load_gather(b3, [row, cols[3] + d])
                rn = plsc.load_gather(b4, [row, cols[4] + d])
                tn = plsc.load_gather(b5, [row, cols[5] + d])
                acc_n = acc_n + jnp.abs(hn + rn - tn)
            hinge = jnp.maximum(acc_p - acc_n + jnp.float32(_MARGIN), zero)
            return loss_sum + hinge

        loss = lax.fori_loop(0, _NVC, vreg_body, loss)

    loss_v[...] = loss
    pltpu.sync_copy(loss_v, out_hbm.at[wid])


_NE = 1000000
_RB = 32768
_NRB = -(-_NE // _RB)
_NPR = _NRB * (_RB // 4)


def _relayout_body(in_ref, out_ref):
    x = in_ref[...].astype(jnp.bfloat16)
    eye = jnp.eye(_D, dtype=jnp.bfloat16)
    y = lax.dot_general(x, eye, (((0,), (0,)), ((), ())),
                        preferred_element_type=jnp.float32)
    for k in range(_RB // 512):
        for q in range(4):
            out_ref[k * 128:(k + 1) * 128, q * _D:(q + 1) * _D] = (
                y[k * 512 + q * 128:k * 512 + (q + 1) * 128, :]
            )


def _pack_rows(table_t):
    return pl.pallas_call(
        _relayout_body,
        grid=(_NRB,),
        in_specs=[pl.BlockSpec((_D, _RB), lambda i: (0, i))],
        out_specs=pl.BlockSpec((_RB // 4, 4 * _D), lambda i: (i, 0)),
        out_shape=jax.ShapeDtypeStruct((_NPR, 4 * _D), jnp.float32),
    )(table_t)


def kernel(positive_triplets, negative_triplets, entity_emb, relation_emb):
    packed = []
    offs = []
    for arr in (positive_triplets, negative_triplets):
        for c in range(3):
            col = arr[:, c]
            packed.append(((col >> 9) * 128 + (col & 127)).reshape(_NW, _NCH, _CH))
            offs.append((((col >> 7) & 3) * _D).reshape(_NW, _BW))
    partials = _transe_sc(
        _pack_rows(entity_emb.T),
        _pack_rows(relation_emb.T),
        *packed, *offs,
    )
    return jnp.sum(partials) / jnp.float32(_B)


# speedup vs baseline: 5.3906x; 1.0155x over previous
import functools

import jax
import jax.numpy as jnp
from jax import lax
from jax.experimental import pallas as pl
from jax.experimental.pallas import tpu as pltpu
from jax.experimental.pallas import tpu_sc as plsc

_D = 32
_B = 16384
_MARGIN = 1.0
_L = 16
_NW = 32
_BW = _B // _NW
_CH = 128
_NCH = _BW // _CH
_NVC = _CH // _L

_mesh = plsc.VectorSubcoreMesh(
    core_axis_name="c", subcore_axis_name="s", num_cores=2, num_subcores=16
)


@functools.partial(
    pl.kernel,
    out_type=jax.ShapeDtypeStruct((_NW, _L), jnp.float32),
    mesh=_mesh,
    scratch_types=(
        [pltpu.VMEM((_NCH, _CH), jnp.int32) for _ in range(6)]
        + [pltpu.VMEM((_BW,), jnp.int32) for _ in range(6)]
        + [pltpu.VMEM((_CH, _CH), jnp.float32) for _ in range(6)]
        + [pltpu.VMEM((_L,), jnp.float32), pltpu.SemaphoreType.DMA]
    ),
    compiler_params=pltpu.CompilerParams(
        needs_layout_passes=False, use_tc_tiling_on_sc=True
    ),
)
def _transe_sc(ent_hbm, rel_hbm,
               p0, p1, p2, p3, p4, p5,
               s0, s1, s2, s3, s4, s5,
               out_hbm,
               ip0, ip1, ip2, ip3, ip4, ip5,
               is0, is1, is2, is3, is4, is5,
               b0, b1, b2, b3, b4, b5,
               loss_v, sem):
    wid = lax.axis_index("s") * 2 + lax.axis_index("c")

    p_hbms = (p0, p1, p2, p3, p4, p5)
    s_hbms = (s0, s1, s2, s3, s4, s5)
    ip_refs = (ip0, ip1, ip2, ip3, ip4, ip5)
    is_refs = (is0, is1, is2, is3, is4, is5)
    bufs = (b0, b1, b2, b3, b4, b5)
    tables = (ent_hbm, rel_hbm, ent_hbm, ent_hbm, rel_hbm, ent_hbm)

    copies = [pltpu.async_copy(h.at[wid], r, sem)
              for h, r in zip(p_hbms + s_hbms, ip_refs + is_refs)]
    for c in copies:
        c.wait()

    lane = lax.iota(jnp.int32, _L)
    zero = lax.broadcast(jnp.float32(0.0), (_L,))
    loss = zero

    for c in range(_NCH):
        gathers = [
            pltpu.async_copy(tab.at[iref.at[c]], bref, sem)
            for tab, iref, bref in zip(tables, ip_refs, bufs)
        ]
        for g in gathers:
            g.wait()

        def vreg_body(v, loss_sum, _c=c):
            row = lane + v * _L
            off = _c * _CH
            cols = [plsc.load_gather(sref, [row + off]) for sref in is_refs]
            acc_p = zero
            acc_n = zero
            for d in range(_D):
                hp = plsc.load_gather(b0, [row, cols[0] + d])
                rp = plsc.load_gather(b1, [row, cols[1] + d])
                tp = plsc.load_gather(b2, [row, cols[2] + d])
                acc_p = acc_p + jnp.abs(hp + rp - tp)
                hn = plsc.load_gather(b3, [row, cols[3] + d])
                rn = plsc.load_gather(b4, [row, cols[4] + d])
                tn = plsc.load_gather(b5, [row, cols[5] + d])
                acc_n = acc_n + jnp.abs(hn + rn - tn)
            hinge = jnp.maximum(acc_p - acc_n + jnp.float32(_MARGIN), zero)
            return loss_sum + hinge

        loss = lax.fori_loop(0, _NVC, vreg_body, loss)

    loss_v[...] = loss
    pltpu.sync_copy(loss_v, out_hbm.at[wid])


_NE = 1000000
_RB = 32768
_NRB = -(-_NE // _RB)
_NPR = _NRB * (_RB // 4)


def _relayout_body(ent_ref, rel_ref, out_e_ref, out_r_ref):
    eye = jnp.eye(_D, dtype=jnp.bfloat16)
    for in_ref, out_ref in ((ent_ref, out_e_ref), (rel_ref, out_r_ref)):
        x = in_ref[...].astype(jnp.bfloat16)
        y = lax.dot_general(x, eye, (((0,), (0,)), ((), ())),
                            preferred_element_type=jnp.float32)
        for k in range(_RB // 512):
            for q in range(4):
                out_ref[k * 128:(k + 1) * 128, q * _D:(q + 1) * _D] = (
                    y[k * 512 + q * 128:k * 512 + (q + 1) * 128, :]
                )


def _pack_rows(ent_t, rel_t):
    spec_in = pl.BlockSpec((_D, _RB), lambda i: (0, i))
    spec_out = pl.BlockSpec((_RB // 4, 4 * _D), lambda i: (i, 0))
    out_ty = jax.ShapeDtypeStruct((_NPR, 4 * _D), jnp.float32)
    return pl.pallas_call(
        _relayout_body,
        grid=(_NRB,),
        in_specs=[spec_in, spec_in],
        out_specs=[spec_out, spec_out],
        out_shape=[out_ty, out_ty],
    )(ent_t, rel_t)


def kernel(positive_triplets, negative_triplets, entity_emb, relation_emb):
    packed = []
    offs = []
    for arr in (positive_triplets, negative_triplets):
        for c in range(3):
            col = arr[:, c]
            packed.append(((col >> 9) * 128 + (col & 127)).reshape(_NW, _NCH, _CH))
            offs.append((((col >> 7) & 3) * _D).reshape(_NW, _BW))
    ent_pk, rel_pk = _pack_rows(entity_emb.T, relation_emb.T)
    partials = _transe_sc(ent_pk, rel_pk, *packed, *offs)
    return jnp.sum(partials) / jnp.float32(_B)
